# baseline (speedup 1.0000x reference)
import functools

import numpy as np
import jax
import jax.numpy as jnp
from jax import lax
from jax.experimental import pallas as pl
from jax.experimental.pallas import tpu as pltpu
from jax.experimental.pallas import tpu_sc as plsc

DIM = 128
BATCH = 512
NEG_DIST = 2.0

NC = 2
NS = 16
NW = NC * NS

N_CSEC = 15
N_RSEC = 4
CLS_B = N_CSEC * BATCH
REL_B = N_RSEC * BATCH

_TBL_COLS = (2, 3, 3, 3, 2, 3, 3)
_GROUPS = (
    (0, 0, 'c', 0, 0), (0, 0, 'c', 1, 1),
    (1, 1, 'c', 0, 2), (1, 1, 'c', 1, 3), (1, 1, 'c', 2, 4),
    (2, 2, 'c', 0, 5), (2, 2, 'c', 2, 6), (2, 2, 'r', 1, 0),
    (3, 3, 'c', 1, 7), (3, 3, 'c', 2, 8), (3, 3, 'r', 0, 1),
    (4, 4, 'c', 0, 9), (4, 4, 'c', 1, 10),
    (5, 5, 'c', 0, 11), (5, 5, 'c', 2, 13), (5, 5, 'r', 1, 2),
    (6, 5, 'c', 0, 12), (6, 5, 'c', 2, 14), (6, 5, 'r', 1, 3),
)
NG = len(_GROUPS)

_CONST_CACHE = {}


def _index_consts():
    if 'pos' not in _CONST_CACHE:
        kk = jax.random.split(jax.random.key(42), 6)
        iv = [np.asarray(jax.random.randint(kk[t], (BATCH,), 0, 100000))
              for t in range(6)]
        pos = np.zeros((NW, NG * 16), np.int32)
        for g, (t, i, _kind, col, _sec) in enumerate(_GROUPS):
            p = iv[i] * _TBL_COLS[t] + col
            pos[:, g * 16:(g + 1) * 16] = p.reshape(NW, 16)
        _CONST_CACHE['pos'] = pos
    return _CONST_CACHE['pos']


def _gather_body(t0, t1, t2, t3, t4, t5, t6, ce, re, pos_c,
                 cls_out, rel_out,
                 pos_v, cidx_v, ridx_v, crows_v, rrows_v, sem):
    wid = lax.axis_index("s") * NC + lax.axis_index("c")
    pltpu.sync_copy(pos_c.at[wid], pos_v)
    tabs = (t0, t1, t2, t3, t4, t5, t6)
    descs = []
    for g, (t, _i, kind, _col, sec) in enumerate(_GROUPS):
        dst = (cidx_v if kind == 'c' else ridx_v).at[pl.ds(sec * 16, 16)]
        descs.append(pltpu.async_copy(
            tabs[t].at[pos_v.at[pl.ds(g * 16, 16)]], dst, sem))
    for d in descs:
        d.wait()
    half = (N_CSEC * 16) // 2
    for j in range(2):
        pltpu.async_copy(ce.at[cidx_v.at[pl.ds(j * half, half)]],
                         crows_v.at[pl.ds(j * half, half)], sem).wait()
    pltpu.async_copy(re.at[ridx_v], rrows_v, sem).wait()
    for s in range(N_CSEC):
        pltpu.sync_copy(crows_v.at[pl.ds(s * 16, 16)],
                        cls_out.at[pl.ds(s * BATCH + wid * 16, 16)])
    for r in range(N_RSEC):
        pltpu.sync_copy(rrows_v.at[pl.ds(r * 16, 16)],
                        rel_out.at[pl.ds(r * BATCH + wid * 16, 16)])


def _gather_sc(tabs, class_embeds, relation_embeds, pos_c):
    mesh = plsc.VectorSubcoreMesh(core_axis_name="c", subcore_axis_name="s",
                                  num_cores=NC, num_subcores=NS)
    f = pl.kernel(
        _gather_body,
        out_type=[
            jax.ShapeDtypeStruct((CLS_B, 2 * DIM), jnp.float32),
            jax.ShapeDtypeStruct((REL_B, DIM), jnp.float32),
        ],
        mesh=mesh,
        scratch_types=[
            pltpu.VMEM((NG * 16,), jnp.int32),
            pltpu.VMEM((N_CSEC * 16,), jnp.int32),
            pltpu.VMEM((N_RSEC * 16,), jnp.int32),
            pltpu.VMEM((N_CSEC * 16, 2 * DIM), jnp.float32),
            pltpu.VMEM((N_RSEC * 16, DIM), jnp.float32),
            pltpu.SemaphoreType.DMA,
        ],
    )
    return f(*tabs, class_embeds, relation_embeds, pos_c)


def _ssq_relu(x):
    return jnp.sum(jnp.square(jax.nn.relu(x)), axis=1)


def _loss_body(cls_ref, rel_ref, out_ref):
    e = cls_ref[...]
    c = e[:, :DIM]
    o = jnp.abs(e[:, DIM:])
    rel = rel_ref[...]
    S = BATCH

    def cs(k, n=1):
        return c[k * S:(k + n) * S], o[k * S:(k + n) * S]

    cc, co = cs(0)
    dc, do = cs(1)
    loss1 = jnp.mean(_ssq_relu(jnp.abs(cc - dc) + co - do))

    cc, co = cs(2)
    dc, do = cs(3)
    ec, eo = cs(4)
    lower = jnp.maximum(cc - co, dc - do)
    upper = jnp.minimum(cc + co, dc + do)
    ic = (lower + upper) * 0.5
    io = jnp.abs(upper - lower) * 0.5
    a = jnp.sqrt(_ssq_relu(jnp.abs(ic - ec) + io - eo))
    b = jnp.sqrt(_ssq_relu(lower - upper))
    loss2 = (jnp.mean(jnp.square(a)) + jnp.mean(jnp.square(b))
             + 2.0 * jnp.mean(a) * jnp.mean(b))

    cc, co = cs(5)
    dc, do = cs(6)
    r = rel[0 * S:1 * S]
    loss3 = jnp.mean(_ssq_relu(jnp.abs(cc + r - dc) + co - do))

    cc, co = cs(7)
    dc, do = cs(8)
    r = rel[1 * S:2 * S]
    loss4 = jnp.mean(_ssq_relu(jnp.abs(cc - r - dc) + co - do))

    cc, co = cs(9)
    dc, do = cs(10)
    loss_dis = jnp.mean(_ssq_relu(-jnp.abs(cc - dc) + co + do))

    cc, co = cs(11, 2)
    dc, do = cs(13, 2)
    r = rel[2 * S:4 * S]
    nneg = jnp.sqrt(_ssq_relu(jnp.abs(cc + r - dc) - co - do))
    neg_loss = jnp.mean(jnp.square(NEG_DIST - nneg))

    total = loss1 + loss2 + loss_dis + loss3 + loss4 + neg_loss
    out_ref[...] = jnp.reshape(total, (1, 1))


def _loss_tc(cls_rows, rel_rows):
    return pl.pallas_call(
        _loss_body,
        out_shape=jax.ShapeDtypeStruct((1, 1), jnp.float32),
    )(cls_rows, rel_rows)


_POS_C = _index_consts()


def kernel(nf1, nf2, nf3, nf4, disjoint, nf3_neg0, nf3_neg1,
           class_embeds, relation_embeds):
    tabs = [t.reshape(-1) for t in
            (nf1, nf2, nf3, nf4, disjoint, nf3_neg0, nf3_neg1)]
    cls_rows, rel_rows = _gather_sc(tabs, class_embeds, relation_embeds,
                                    jnp.asarray(_POS_C))
    return _loss_tc(cls_rows, rel_rows)[0, 0]


# speedup vs baseline: 1.4108x; 1.4108x over previous
import numpy as np
import jax
import jax.numpy as jnp
from jax import lax
from jax.experimental import pallas as pl
from jax.experimental.pallas import tpu as pltpu
from jax.experimental.pallas import tpu_sc as plsc

DIM = 128
BATCH = 512
NEG_DIST = 2.0

NC = 2
NS = 16
NW = NC * NS

N_CSEC = 15
N_RSEC = 4
CLS_B = N_CSEC * BATCH
REL_B = N_RSEC * BATCH

_TBL_COLS = (2, 3, 3, 3, 2, 3, 3)
_GROUPS = (
    (0, 'c', 0, 0), (0, 'c', 1, 1),
    (1, 'c', 0, 2), (1, 'c', 1, 3), (1, 'c', 2, 4),
    (2, 'c', 0, 5), (2, 'c', 2, 6), (2, 'r', 1, 0),
    (3, 'c', 1, 7), (3, 'c', 2, 8), (3, 'r', 0, 1),
    (4, 'c', 0, 9), (4, 'c', 1, 10),
    (5, 'c', 0, 11), (5, 'c', 2, 13), (5, 'r', 1, 2),
    (6, 'c', 0, 12), (6, 'c', 2, 14), (6, 'r', 1, 3),
)


_U32 = np.uint32


def _rol(x, d):
    return (x << _U32(d)) | (x >> _U32(32 - d))


def _threefry2x32(k1, k2, x1, x2):
    ks0, ks1 = _U32(k1), _U32(k2)
    ks2 = ks0 ^ ks1 ^ _U32(0x1BD11BDA)
    with np.errstate(over='ignore'):
        x0 = (np.asarray(x1, _U32) + ks0).astype(_U32)
        y1 = (np.asarray(x2, _U32) + ks1).astype(_U32)

        def rounds(x0, y1, rots):
            for r in rots:
                x0 = (x0 + y1).astype(_U32)
                y1 = x0 ^ _rol(y1, r)
            return x0, y1

        x0, y1 = rounds(x0, y1, (13, 15, 26, 6))
        x0 = (x0 + ks1).astype(_U32); y1 = (y1 + ks2 + _U32(1)).astype(_U32)
        x0, y1 = rounds(x0, y1, (17, 29, 16, 24))
        x0 = (x0 + ks2).astype(_U32); y1 = (y1 + ks0 + _U32(2)).astype(_U32)
        x0, y1 = rounds(x0, y1, (13, 15, 26, 6))
        x0 = (x0 + ks0).astype(_U32); y1 = (y1 + ks1 + _U32(3)).astype(_U32)
        x0, y1 = rounds(x0, y1, (17, 29, 16, 24))
        x0 = (x0 + ks1).astype(_U32); y1 = (y1 + ks2 + _U32(4)).astype(_U32)
        x0, y1 = rounds(x0, y1, (13, 15, 26, 6))
        x0 = (x0 + ks2).astype(_U32); y1 = (y1 + ks0 + _U32(5)).astype(_U32)
    return x0, y1


def _np_split(key, num):
    b1, b2 = _threefry2x32(key[0], key[1],
                           np.zeros(num, _U32), np.arange(num, dtype=_U32))
    return [(b1[i], b2[i]) for i in range(num)]


def _np_bits(key, n):
    b1, b2 = _threefry2x32(key[0], key[1],
                           np.zeros(n, _U32), np.arange(n, dtype=_U32))
    return b1 ^ b2


def _np_randint(key, n, minval, maxval):
    k1, k2 = _np_split(key, 2)
    higher, lower = _np_bits(k1, n), _np_bits(k2, n)
    span = _U32(maxval - minval)
    with np.errstate(over='ignore'):
        mult = _U32(int(2 ** 16) % int(span))
        mult = _U32((int(mult) * int(mult)) % (2 ** 32)) % span
        off = ((higher % span) * mult + lower % span).astype(_U32) % span
    return (np.int32(minval) + off.astype(np.int32)).astype(np.int32)


def _batch_index_consts():
    keys = _np_split((_U32(0), _U32(42)), 6)
    return [_np_randint(keys[t], BATCH, 0, 100000) for t in range(6)]


_IVECS = _batch_index_consts()


def _gather_body(d0, d1, d2, d3, d4, d5, d6, ce, re,
                 cls_out, rel_out,
                 dv, cidx_v, ridx_v, crows_v, rrows_v, sem):
    wid = lax.axis_index("s") * NC + lax.axis_index("c")
    flats = (d0, d1, d2, d3, d4, d5, d6)
    offs = []
    off = 0
    for t, f in enumerate(flats):
        n = 16 * _TBL_COLS[t]
        pltpu.sync_copy(f.at[pl.ds(wid * n, n)], dv.at[pl.ds(off, n)])
        offs.append(off)
        off += n
    iota = lax.iota(jnp.int32, 16)
    for t, kind, col, sec in _GROUPS:
        idx16 = plsc.load_gather(dv, [iota * _TBL_COLS[t]
                                      + (offs[t] + col)])
        if kind == 'c':
            cidx_v[pl.ds(sec * 16, 16)] = idx16
        else:
            ridx_v[pl.ds(sec * 16, 16)] = idx16
    half = (N_CSEC * 16) // 2
    for j in range(2):
        pltpu.async_copy(ce.at[cidx_v.at[pl.ds(j * half, half)]],
                         crows_v.at[pl.ds(j * half, half)], sem).wait()
    pltpu.async_copy(re.at[ridx_v], rrows_v, sem).wait()
    for s in range(N_CSEC):
        pltpu.sync_copy(crows_v.at[pl.ds(s * 16, 16)],
                        cls_out.at[pl.ds(s * BATCH + wid * 16, 16)])
    for r in range(N_RSEC):
        pltpu.sync_copy(rrows_v.at[pl.ds(r * 16, 16)],
                        rel_out.at[pl.ds(r * BATCH + wid * 16, 16)])


def _gather_sc(flats, class_embeds, relation_embeds):
    mesh = plsc.VectorSubcoreMesh(core_axis_name="c", subcore_axis_name="s",
                                  num_cores=NC, num_subcores=NS)
    f = pl.kernel(
        _gather_body,
        out_type=[
            jax.ShapeDtypeStruct((CLS_B, 2 * DIM), jnp.float32),
            jax.ShapeDtypeStruct((REL_B, DIM), jnp.float32),
        ],
        mesh=mesh,
        compiler_params=pltpu.CompilerParams(needs_layout_passes=False),
        scratch_types=[
            pltpu.VMEM((16 * sum(_TBL_COLS),), jnp.int32),
            pltpu.VMEM((N_CSEC * 16,), jnp.int32),
            pltpu.VMEM((N_RSEC * 16,), jnp.int32),
            pltpu.VMEM((N_CSEC * 16, 2 * DIM), jnp.float32),
            pltpu.VMEM((N_RSEC * 16, DIM), jnp.float32),
            pltpu.SemaphoreType.DMA,
        ],
    )
    return f(*flats, class_embeds, relation_embeds)


def _ssq_relu(x):
    return jnp.sum(jnp.square(jax.nn.relu(x)), axis=1)


def _loss_body(cls_ref, rel_ref, out_ref):
    e = cls_ref[...]
    c = e[:, :DIM]
    o = jnp.abs(e[:, DIM:])
    rel = rel_ref[...]
    S = BATCH

    def cs(k, n=1):
        return c[k * S:(k + n) * S], o[k * S:(k + n) * S]

    cc, co = cs(0)
    dc, do = cs(1)
    loss1 = jnp.mean(_ssq_relu(jnp.abs(cc - dc) + co - do))

    cc, co = cs(2)
    dc, do = cs(3)
    ec, eo = cs(4)
    lower = jnp.maximum(cc - co, dc - do)
    upper = jnp.minimum(cc + co, dc + do)
    ic = (lower + upper) * 0.5
    io = jnp.abs(upper - lower) * 0.5
    a = jnp.sqrt(_ssq_relu(jnp.abs(ic - ec) + io - eo))
    b = jnp.sqrt(_ssq_relu(lower - upper))
    loss2 = (jnp.mean(jnp.square(a)) + jnp.mean(jnp.square(b))
             + 2.0 * jnp.mean(a) * jnp.mean(b))

    cc, co = cs(5)
    dc, do = cs(6)
    r = rel[0 * S:1 * S]
    loss3 = jnp.mean(_ssq_relu(jnp.abs(cc + r - dc) + co - do))

    cc, co = cs(7)
    dc, do = cs(8)
    r = rel[1 * S:2 * S]
    loss4 = jnp.mean(_ssq_relu(jnp.abs(cc - r - dc) + co - do))

    cc, co = cs(9)
    dc, do = cs(10)
    loss_dis = jnp.mean(_ssq_relu(-jnp.abs(cc - dc) + co + do))

    cc, co = cs(11, 2)
    dc, do = cs(13, 2)
    r = rel[2 * S:4 * S]
    nneg = jnp.sqrt(_ssq_relu(jnp.abs(cc + r - dc) - co - do))
    neg_loss = jnp.mean(jnp.square(NEG_DIST - nneg))

    total = loss1 + loss2 + loss_dis + loss3 + loss4 + neg_loss
    out_ref[...] = jnp.reshape(total, (1, 1))


def _loss_tc(cls_rows, rel_rows):
    return pl.pallas_call(
        _loss_body,
        out_shape=jax.ShapeDtypeStruct((1, 1), jnp.float32),
    )(cls_rows, rel_rows)


def kernel(nf1, nf2, nf3, nf4, disjoint, nf3_neg0, nf3_neg1,
           class_embeds, relation_embeds):
    tables = (nf1, nf2, nf3, nf4, disjoint, nf3_neg0, nf3_neg1)
    flats = [tables[t][_IVECS[(5 if t == 6 else t)]].reshape(-1)
             for t in range(7)]
    cls_rows, rel_rows = _gather_sc(flats, class_embeds, relation_embeds)
    return _loss_tc(cls_rows, rel_rows)[0, 0]


# speedup vs baseline: 1.4269x; 1.0114x over previous
import numpy as np
import jax
import jax.numpy as jnp
from jax import lax
from jax.experimental import pallas as pl
from jax.experimental.pallas import tpu as pltpu
from jax.experimental.pallas import tpu_sc as plsc

DIM = 128
BATCH = 512
NEG_DIST = 2.0

NC = 2
NS = 16
NW = NC * NS

N_CSEC = 15
N_RSEC = 4
NCH = (2 * DIM) // 16

_TBL_COLS = (2, 3, 3, 3, 2, 3, 3)
_GROUPS = (
    (0, 'c', 0, 0), (0, 'c', 1, 1),
    (1, 'c', 0, 2), (1, 'c', 1, 3), (1, 'c', 2, 4),
    (2, 'c', 0, 5), (2, 'c', 2, 6), (2, 'r', 1, 0),
    (3, 'c', 1, 7), (3, 'c', 2, 8), (3, 'r', 0, 1),
    (4, 'c', 0, 9), (4, 'c', 1, 10),
    (5, 'c', 0, 11), (5, 'c', 2, 13), (5, 'r', 1, 2),
    (6, 'c', 0, 12), (6, 'c', 2, 14), (6, 'r', 1, 3),
)


_U32 = np.uint32


def _rol(x, d):
    return (x << _U32(d)) | (x >> _U32(32 - d))


def _threefry2x32(k1, k2, x1, x2):
    ks0, ks1 = _U32(k1), _U32(k2)
    ks2 = ks0 ^ ks1 ^ _U32(0x1BD11BDA)
    with np.errstate(over='ignore'):
        x0 = (np.asarray(x1, _U32) + ks0).astype(_U32)
        y1 = (np.asarray(x2, _U32) + ks1).astype(_U32)

        def rounds(x0, y1, rots):
            for r in rots:
                x0 = (x0 + y1).astype(_U32)
                y1 = x0 ^ _rol(y1, r)
            return x0, y1

        x0, y1 = rounds(x0, y1, (13, 15, 26, 6))
        x0 = (x0 + ks1).astype(_U32); y1 = (y1 + ks2 + _U32(1)).astype(_U32)
        x0, y1 = rounds(x0, y1, (17, 29, 16, 24))
        x0 = (x0 + ks2).astype(_U32); y1 = (y1 + ks0 + _U32(2)).astype(_U32)
        x0, y1 = rounds(x0, y1, (13, 15, 26, 6))
        x0 = (x0 + ks0).astype(_U32); y1 = (y1 + ks1 + _U32(3)).astype(_U32)
        x0, y1 = rounds(x0, y1, (17, 29, 16, 24))
        x0 = (x0 + ks1).astype(_U32); y1 = (y1 + ks2 + _U32(4)).astype(_U32)
        x0, y1 = rounds(x0, y1, (13, 15, 26, 6))
        x0 = (x0 + ks2).astype(_U32); y1 = (y1 + ks0 + _U32(5)).astype(_U32)
    return x0, y1


def _np_split(key, num):
    b1, b2 = _threefry2x32(key[0], key[1],
                           np.zeros(num, _U32), np.arange(num, dtype=_U32))
    return [(b1[i], b2[i]) for i in range(num)]


def _np_bits(key, n):
    b1, b2 = _threefry2x32(key[0], key[1],
                           np.zeros(n, _U32), np.arange(n, dtype=_U32))
    return b1 ^ b2


def _np_randint(key, n, minval, maxval):
    k1, k2 = _np_split(key, 2)
    higher, lower = _np_bits(k1, n), _np_bits(k2, n)
    span = _U32(maxval - minval)
    with np.errstate(over='ignore'):
        mult = _U32(int(2 ** 16) % int(span))
        mult = _U32((int(mult) * int(mult)) % (2 ** 32)) % span
        off = ((higher % span) * mult + lower % span).astype(_U32) % span
    return (np.int32(minval) + off.astype(np.int32)).astype(np.int32)


_IVECS = [_np_randint(k, BATCH, 0, 100000)
          for k in _np_split((_U32(0), _U32(42)), 6)]


def _gather_body(d0, d1, d2, d3, d4, d5, d6, ce, re,
                 lin_out, a2_out, b2_out, neg_out,
                 dv, cidx_v, ridx_v, crows_v, rrows_v,
                 lin_v, a2_v, b2_v, neg_v, sem):
    wid = lax.axis_index("s") * NC + lax.axis_index("c")
    flats = (d0, d1, d2, d3, d4, d5, d6)
    offs = []
    off = 0
    for t, f in enumerate(flats):
        n = 16 * _TBL_COLS[t]
        pltpu.sync_copy(f.at[pl.ds(wid * n, n)], dv.at[pl.ds(off, n)])
        offs.append(off)
        off += n
    iota = lax.iota(jnp.int32, 16)
    for t, kind, col, sec in _GROUPS:
        idx16 = plsc.load_gather(dv, [iota * _TBL_COLS[t] + (offs[t] + col)])
        if kind == 'c':
            cidx_v[pl.ds(sec * 16, 16)] = idx16
        else:
            ridx_v[pl.ds(sec * 16, 16)] = idx16
    half = (N_CSEC * 16) // 2
    for j in range(2):
        pltpu.async_copy(ce.at[cidx_v.at[pl.ds(j * half, half)]],
                         crows_v.at[pl.ds(j * half, half)], sem).wait()
    pltpu.async_copy(re.at[ridx_v], rrows_v, sem).wait()

    def crow(sec, j, ch):
        return crows_v[sec * 16 + j, pl.ds(ch * 16, 16)]

    def rrow(sec, j, ch):
        return rrows_v[sec * 16 + j, pl.ds(ch * 16, 16)]

    def relu(x):
        return jnp.maximum(x, 0.0)

    def body(j, lin_acc):
        zero = jnp.zeros((16,), jnp.float32)
        for ch in range(8):
            cc, co = crow(0, j, ch), jnp.abs(crow(0, j, ch + 8))
            dc, do = crow(1, j, ch), jnp.abs(crow(1, j, ch + 8))
            t1 = relu(jnp.abs(cc - dc) + co - do)
            cc, co = crow(5, j, ch), jnp.abs(crow(5, j, ch + 8))
            dc, do = crow(6, j, ch), jnp.abs(crow(6, j, ch + 8))
            t3 = relu(jnp.abs(cc + rrow(0, j, ch) - dc) + co - do)
            cc, co = crow(7, j, ch), jnp.abs(crow(7, j, ch + 8))
            dc, do = crow(8, j, ch), jnp.abs(crow(8, j, ch + 8))
            t4 = relu(jnp.abs(cc - rrow(1, j, ch) - dc) + co - do)
            cc, co = crow(9, j, ch), jnp.abs(crow(9, j, ch + 8))
            dc, do = crow(10, j, ch), jnp.abs(crow(10, j, ch + 8))
            td = relu(-jnp.abs(cc - dc) + co + do)
            lin_acc = lin_acc + t1 * t1 + t3 * t3 + t4 * t4 + td * td
        a2 = zero
        b2 = zero
        for ch in range(8):
            cc, co = crow(2, j, ch), jnp.abs(crow(2, j, ch + 8))
            dc, do = crow(3, j, ch), jnp.abs(crow(3, j, ch + 8))
            ec, eo = crow(4, j, ch), jnp.abs(crow(4, j, ch + 8))
            lower = jnp.maximum(cc - co, dc - do)
            upper = jnp.minimum(cc + co, dc + do)
            ic = (lower + upper) * 0.5
            io = jnp.abs(upper - lower) * 0.5
            ta = relu(jnp.abs(ic - ec) + io - eo)
            tb = relu(lower - upper)
            a2 = a2 + ta * ta
            b2 = b2 + tb * tb
        a2_v[j, pl.ds(0, 16)] = a2
        b2_v[j, pl.ds(0, 16)] = b2
        for h, (sc_, sd, sr) in enumerate(((11, 13, 2), (12, 14, 3))):
            n2 = zero
            for ch in range(8):
                cc, co = crow(sc_, j, ch), jnp.abs(crow(sc_, j, ch + 8))
                dc, do = crow(sd, j, ch), jnp.abs(crow(sd, j, ch + 8))
                tn = relu(jnp.abs(cc + rrow(sr, j, ch) - dc) - co - do)
                n2 = n2 + tn * tn
            neg_v[h * 16 + j, pl.ds(0, 16)] = n2
        return lin_acc

    lin_acc = lax.fori_loop(0, 16, body, jnp.zeros((16,), jnp.float32))
    lin_v[pl.ds(0, 16)] = lin_acc

    pltpu.sync_copy(lin_v, lin_out.at[wid])
    pltpu.sync_copy(a2_v, a2_out.at[wid])
    pltpu.sync_copy(b2_v, b2_out.at[wid])
    pltpu.sync_copy(neg_v, neg_out.at[wid])


def _gather_sc(flats, class_embeds, relation_embeds):
    mesh = plsc.VectorSubcoreMesh(core_axis_name="c", subcore_axis_name="s",
                                  num_cores=NC, num_subcores=NS)
    f = pl.kernel(
        _gather_body,
        out_type=[
            jax.ShapeDtypeStruct((NW, 16), jnp.float32),
            jax.ShapeDtypeStruct((NW, 16, 16), jnp.float32),
            jax.ShapeDtypeStruct((NW, 16, 16), jnp.float32),
            jax.ShapeDtypeStruct((NW, 32, 16), jnp.float32),
        ],
        mesh=mesh,
        compiler_params=pltpu.CompilerParams(needs_layout_passes=False),
        scratch_types=[
            pltpu.VMEM((16 * sum(_TBL_COLS),), jnp.int32),
            pltpu.VMEM((N_CSEC * 16,), jnp.int32),
            pltpu.VMEM((N_RSEC * 16,), jnp.int32),
            pltpu.VMEM((N_CSEC * 16, 2 * DIM), jnp.float32),
            pltpu.VMEM((N_RSEC * 16, DIM), jnp.float32),
            pltpu.VMEM((16,), jnp.float32),
            pltpu.VMEM((16, 16), jnp.float32),
            pltpu.VMEM((16, 16), jnp.float32),
            pltpu.VMEM((32, 16), jnp.float32),
            pltpu.SemaphoreType.DMA,
        ],
    )
    return f(*flats, class_embeds, relation_embeds)


def _epi_body(lin_ref, a2_ref, b2_ref, neg_ref, out_ref):
    loss_lin = jnp.sum(lin_ref[...]) / BATCH
    a2 = jnp.sum(a2_ref[...], axis=1)
    b2 = jnp.sum(b2_ref[...], axis=1)
    a = jnp.sqrt(a2)
    b = jnp.sqrt(b2)
    loss2 = (jnp.mean(a2) + jnp.mean(b2) + 2.0 * jnp.mean(a) * jnp.mean(b))
    n2 = jnp.sum(neg_ref[...], axis=1)
    n = jnp.sqrt(n2)
    neg_loss = jnp.mean(jnp.square(NEG_DIST - n))
    total = loss_lin + loss2 + neg_loss
    out_ref[...] = jnp.reshape(total, (1, 1))


def _epilogue_tc(lin, a2, b2, neg):
    return pl.pallas_call(
        _epi_body,
        out_shape=jax.ShapeDtypeStruct((1, 1), jnp.float32),
    )(lin, a2.reshape(NW * 16, 16), b2.reshape(NW * 16, 16),
      neg.reshape(NW * 32, 16))


def kernel(nf1, nf2, nf3, nf4, disjoint, nf3_neg0, nf3_neg1,
           class_embeds, relation_embeds):
    tables = (nf1, nf2, nf3, nf4, disjoint, nf3_neg0, nf3_neg1)
    flats = [tables[t][_IVECS[(5 if t == 6 else t)]].reshape(-1)
             for t in range(7)]
    lin, a2, b2, neg = _gather_sc(flats, class_embeds, relation_embeds)
    return _epilogue_tc(lin, a2, b2, neg)[0, 0]


# speedup vs baseline: 2.6246x; 1.8394x over previous
import numpy as np
import jax
import jax.numpy as jnp
from jax import lax
from jax.experimental import pallas as pl
from jax.experimental.pallas import tpu as pltpu
from jax.experimental.pallas import tpu_sc as plsc

DIM = 128
BATCH = 512
NEG_DIST = 2.0

NC = 2
NS = 16
NW = NC * NS

N_CSEC = 15
N_RSEC = 4
NCH = (2 * DIM) // 16

_TBL_COLS = (2, 3, 3, 3, 2, 3, 3)
_GROUPS = (
    (0, 'c', 0, 0), (0, 'c', 1, 1),
    (1, 'c', 0, 2), (1, 'c', 1, 3), (1, 'c', 2, 4),
    (2, 'c', 0, 5), (2, 'c', 2, 6), (2, 'r', 1, 0),
    (3, 'c', 1, 7), (3, 'c', 2, 8), (3, 'r', 0, 1),
    (4, 'c', 0, 9), (4, 'c', 1, 10),
    (5, 'c', 0, 11), (5, 'c', 2, 13), (5, 'r', 1, 2),
    (6, 'c', 0, 12), (6, 'c', 2, 14), (6, 'r', 1, 3),
)


_U32 = np.uint32


def _rol(x, d):
    return (x << _U32(d)) | (x >> _U32(32 - d))


def _threefry2x32(k1, k2, x1, x2):
    ks0, ks1 = _U32(k1), _U32(k2)
    ks2 = ks0 ^ ks1 ^ _U32(0x1BD11BDA)
    with np.errstate(over='ignore'):
        x0 = (np.asarray(x1, _U32) + ks0).astype(_U32)
        y1 = (np.asarray(x2, _U32) + ks1).astype(_U32)

        def rounds(x0, y1, rots):
            for r in rots:
                x0 = (x0 + y1).astype(_U32)
                y1 = x0 ^ _rol(y1, r)
            return x0, y1

        x0, y1 = rounds(x0, y1, (13, 15, 26, 6))
        x0 = (x0 + ks1).astype(_U32); y1 = (y1 + ks2 + _U32(1)).astype(_U32)
        x0, y1 = rounds(x0, y1, (17, 29, 16, 24))
        x0 = (x0 + ks2).astype(_U32); y1 = (y1 + ks0 + _U32(2)).astype(_U32)
        x0, y1 = rounds(x0, y1, (13, 15, 26, 6))
        x0 = (x0 + ks0).astype(_U32); y1 = (y1 + ks1 + _U32(3)).astype(_U32)
        x0, y1 = rounds(x0, y1, (17, 29, 16, 24))
        x0 = (x0 + ks1).astype(_U32); y1 = (y1 + ks2 + _U32(4)).astype(_U32)
        x0, y1 = rounds(x0, y1, (13, 15, 26, 6))
        x0 = (x0 + ks2).astype(_U32); y1 = (y1 + ks0 + _U32(5)).astype(_U32)
    return x0, y1


def _np_split(key, num):
    b1, b2 = _threefry2x32(key[0], key[1],
                           np.zeros(num, _U32), np.arange(num, dtype=_U32))
    return [(b1[i], b2[i]) for i in range(num)]


def _np_bits(key, n):
    b1, b2 = _threefry2x32(key[0], key[1],
                           np.zeros(n, _U32), np.arange(n, dtype=_U32))
    return b1 ^ b2


def _np_randint(key, n, minval, maxval):
    k1, k2 = _np_split(key, 2)
    higher, lower = _np_bits(k1, n), _np_bits(k2, n)
    span = _U32(maxval - minval)
    with np.errstate(over='ignore'):
        mult = _U32(int(2 ** 16) % int(span))
        mult = _U32((int(mult) * int(mult)) % (2 ** 32)) % span
        off = ((higher % span) * mult + lower % span).astype(_U32) % span
    return (np.int32(minval) + off.astype(np.int32)).astype(np.int32)


_IVECS = [_np_randint(k, BATCH, 0, 100000)
          for k in _np_split((_U32(0), _U32(42)), 6)]


def _gather_body(*args):
    (cols, (ce, re), (lin_out, a2_out, b2_out, neg_out),
     (cidx_v, ridx_v, crows_v, rrows_v, lin_v, a2_v, b2_v, neg_v, sem)) = (
        args[:19], args[19:21], args[21:25], args[25:])
    wid = lax.axis_index("s") * NC + lax.axis_index("c")
    descs = []
    for g, (_t, kind, _col, sec) in enumerate(_GROUPS):
        dst = (cidx_v if kind == 'c' else ridx_v).at[pl.ds(sec * 16, 16)]
        descs.append(pltpu.async_copy(
            cols[g].at[pl.ds(wid * 16, 16)], dst, sem))
    for d in descs:
        d.wait()
    half = (N_CSEC * 16) // 2
    for j in range(2):
        pltpu.async_copy(ce.at[cidx_v.at[pl.ds(j * half, half)]],
                         crows_v.at[pl.ds(j * half, half)], sem).wait()
    pltpu.async_copy(re.at[ridx_v], rrows_v, sem).wait()

    def crow(sec, j, ch):
        return crows_v[sec * 16 + j, pl.ds(ch * 16, 16)]

    def rrow(sec, j, ch):
        return rrows_v[sec * 16 + j, pl.ds(ch * 16, 16)]

    def relu(x):
        return jnp.maximum(x, 0.0)

    def body(j, lin_acc):
        zero = jnp.zeros((16,), jnp.float32)
        for ch in range(8):
            cc, co = crow(0, j, ch), jnp.abs(crow(0, j, ch + 8))
            dc, do = crow(1, j, ch), jnp.abs(crow(1, j, ch + 8))
            t1 = relu(jnp.abs(cc - dc) + co - do)
            cc, co = crow(5, j, ch), jnp.abs(crow(5, j, ch + 8))
            dc, do = crow(6, j, ch), jnp.abs(crow(6, j, ch + 8))
            t3 = relu(jnp.abs(cc + rrow(0, j, ch) - dc) + co - do)
            cc, co = crow(7, j, ch), jnp.abs(crow(7, j, ch + 8))
            dc, do = crow(8, j, ch), jnp.abs(crow(8, j, ch + 8))
            t4 = relu(jnp.abs(cc - rrow(1, j, ch) - dc) + co - do)
            cc, co = crow(9, j, ch), jnp.abs(crow(9, j, ch + 8))
            dc, do = crow(10, j, ch), jnp.abs(crow(10, j, ch + 8))
            td = relu(-jnp.abs(cc - dc) + co + do)
            lin_acc = lin_acc + t1 * t1 + t3 * t3 + t4 * t4 + td * td
        a2 = zero
        b2 = zero
        for ch in range(8):
            cc, co = crow(2, j, ch), jnp.abs(crow(2, j, ch + 8))
            dc, do = crow(3, j, ch), jnp.abs(crow(3, j, ch + 8))
            ec, eo = crow(4, j, ch), jnp.abs(crow(4, j, ch + 8))
            lower = jnp.maximum(cc - co, dc - do)
            upper = jnp.minimum(cc + co, dc + do)
            ic = (lower + upper) * 0.5
            io = jnp.abs(upper - lower) * 0.5
            ta = relu(jnp.abs(ic - ec) + io - eo)
            tb = relu(lower - upper)
            a2 = a2 + ta * ta
            b2 = b2 + tb * tb
        a2_v[j, pl.ds(0, 16)] = a2
        b2_v[j, pl.ds(0, 16)] = b2
        for h, (sc_, sd, sr) in enumerate(((11, 13, 2), (12, 14, 3))):
            n2 = zero
            for ch in range(8):
                cc, co = crow(sc_, j, ch), jnp.abs(crow(sc_, j, ch + 8))
                dc, do = crow(sd, j, ch), jnp.abs(crow(sd, j, ch + 8))
                tn = relu(jnp.abs(cc + rrow(sr, j, ch) - dc) - co - do)
                n2 = n2 + tn * tn
            neg_v[h * 16 + j, pl.ds(0, 16)] = n2
        return lin_acc

    lin_acc = lax.fori_loop(0, 16, body, jnp.zeros((16,), jnp.float32))
    lin_v[pl.ds(0, 16)] = lin_acc

    pltpu.sync_copy(lin_v, lin_out.at[wid])
    pltpu.sync_copy(a2_v, a2_out.at[wid])
    pltpu.sync_copy(b2_v, b2_out.at[wid])
    pltpu.sync_copy(neg_v, neg_out.at[wid])


def _gather_sc(cols, class_embeds, relation_embeds):
    mesh = plsc.VectorSubcoreMesh(core_axis_name="c", subcore_axis_name="s",
                                  num_cores=NC, num_subcores=NS)
    f = pl.kernel(
        _gather_body,
        out_type=[
            jax.ShapeDtypeStruct((NW, 16), jnp.float32),
            jax.ShapeDtypeStruct((NW, 16, 16), jnp.float32),
            jax.ShapeDtypeStruct((NW, 16, 16), jnp.float32),
            jax.ShapeDtypeStruct((NW, 32, 16), jnp.float32),
        ],
        mesh=mesh,
        compiler_params=pltpu.CompilerParams(needs_layout_passes=False),
        scratch_types=[
            pltpu.VMEM((N_CSEC * 16,), jnp.int32),
            pltpu.VMEM((N_RSEC * 16,), jnp.int32),
            pltpu.VMEM((N_CSEC * 16, 2 * DIM), jnp.float32),
            pltpu.VMEM((N_RSEC * 16, DIM), jnp.float32),
            pltpu.VMEM((16,), jnp.float32),
            pltpu.VMEM((16, 16), jnp.float32),
            pltpu.VMEM((16, 16), jnp.float32),
            pltpu.VMEM((32, 16), jnp.float32),
            pltpu.SemaphoreType.DMA,
        ],
    )
    return f(*cols, class_embeds, relation_embeds)


def _epi_body(lin_ref, a2_ref, b2_ref, neg_ref, out_ref):
    loss_lin = jnp.sum(lin_ref[...]) / BATCH
    a2 = jnp.sum(a2_ref[...], axis=1)
    b2 = jnp.sum(b2_ref[...], axis=1)
    a = jnp.sqrt(a2)
    b = jnp.sqrt(b2)
    loss2 = (jnp.mean(a2) + jnp.mean(b2) + 2.0 * jnp.mean(a) * jnp.mean(b))
    n2 = jnp.sum(neg_ref[...], axis=1)
    n = jnp.sqrt(n2)
    neg_loss = jnp.mean(jnp.square(NEG_DIST - n))
    total = loss_lin + loss2 + neg_loss
    out_ref[...] = jnp.reshape(total, (1, 1))


def _epilogue_tc(lin, a2, b2, neg):
    return pl.pallas_call(
        _epi_body,
        out_shape=jax.ShapeDtypeStruct((1, 1), jnp.float32),
    )(lin, a2.reshape(NW * 16, 16), b2.reshape(NW * 16, 16),
      neg.reshape(NW * 32, 16))


def kernel(nf1, nf2, nf3, nf4, disjoint, nf3_neg0, nf3_neg1,
           class_embeds, relation_embeds):
    tables = (nf1, nf2, nf3, nf4, disjoint, nf3_neg0, nf3_neg1)
    cols = [tables[t][_IVECS[(5 if t == 6 else t)], col]
            for t, _kind, col, _sec in _GROUPS]
    lin, a2, b2, neg = _gather_sc(cols, class_embeds, relation_embeds)
    return _epilogue_tc(lin, a2, b2, neg)[0, 0]


# speedup vs baseline: 5.3435x; 2.0359x over previous
import numpy as np
import jax
import jax.numpy as jnp
from jax import lax
from jax.experimental import pallas as pl
from jax.experimental.pallas import tpu as pltpu
from jax.experimental.pallas import tpu_sc as plsc

DIM = 128
BATCH = 512
NEG_DIST = 2.0

NC = 2
NS = 16
NW = NC * NS

N_CSEC = 15
N_RSEC = 4
NCH = (2 * DIM) // 16

_TBL_COLS = (2, 3, 3, 3, 2, 3, 3)
_GROUPS = (
    (0, 'c', 0, 0), (0, 'c', 1, 1),
    (1, 'c', 0, 2), (1, 'c', 1, 3), (1, 'c', 2, 4),
    (2, 'c', 0, 5), (2, 'c', 2, 6), (2, 'r', 1, 0),
    (3, 'c', 1, 7), (3, 'c', 2, 8), (3, 'r', 0, 1),
    (4, 'c', 0, 9), (4, 'c', 1, 10),
    (5, 'c', 0, 11), (5, 'c', 2, 13), (5, 'r', 1, 2),
    (6, 'c', 0, 12), (6, 'c', 2, 14), (6, 'r', 1, 3),
)


_U32 = np.uint32


def _rol(x, d):
    return (x << _U32(d)) | (x >> _U32(32 - d))


def _threefry2x32(k1, k2, x1, x2):
    ks0, ks1 = _U32(k1), _U32(k2)
    ks2 = ks0 ^ ks1 ^ _U32(0x1BD11BDA)
    with np.errstate(over='ignore'):
        x0 = (np.asarray(x1, _U32) + ks0).astype(_U32)
        y1 = (np.asarray(x2, _U32) + ks1).astype(_U32)

        def rounds(x0, y1, rots):
            for r in rots:
                x0 = (x0 + y1).astype(_U32)
                y1 = x0 ^ _rol(y1, r)
            return x0, y1

        x0, y1 = rounds(x0, y1, (13, 15, 26, 6))
        x0 = (x0 + ks1).astype(_U32); y1 = (y1 + ks2 + _U32(1)).astype(_U32)
        x0, y1 = rounds(x0, y1, (17, 29, 16, 24))
        x0 = (x0 + ks2).astype(_U32); y1 = (y1 + ks0 + _U32(2)).astype(_U32)
        x0, y1 = rounds(x0, y1, (13, 15, 26, 6))
        x0 = (x0 + ks0).astype(_U32); y1 = (y1 + ks1 + _U32(3)).astype(_U32)
        x0, y1 = rounds(x0, y1, (17, 29, 16, 24))
        x0 = (x0 + ks1).astype(_U32); y1 = (y1 + ks2 + _U32(4)).astype(_U32)
        x0, y1 = rounds(x0, y1, (13, 15, 26, 6))
        x0 = (x0 + ks2).astype(_U32); y1 = (y1 + ks0 + _U32(5)).astype(_U32)
    return x0, y1


def _np_split(key, num):
    b1, b2 = _threefry2x32(key[0], key[1],
                           np.zeros(num, _U32), np.arange(num, dtype=_U32))
    return [(b1[i], b2[i]) for i in range(num)]


def _np_bits(key, n):
    b1, b2 = _threefry2x32(key[0], key[1],
                           np.zeros(n, _U32), np.arange(n, dtype=_U32))
    return b1 ^ b2


def _np_randint(key, n, minval, maxval):
    k1, k2 = _np_split(key, 2)
    higher, lower = _np_bits(k1, n), _np_bits(k2, n)
    span = _U32(maxval - minval)
    with np.errstate(over='ignore'):
        mult = _U32(int(2 ** 16) % int(span))
        mult = _U32((int(mult) * int(mult)) % (2 ** 32)) % span
        off = ((higher % span) * mult + lower % span).astype(_U32) % span
    return (np.int32(minval) + off.astype(np.int32)).astype(np.int32)


_IVECS = [_np_randint(k, BATCH, 0, 100000)
          for k in _np_split((_U32(0), _U32(42)), 6)]


def _gather_body(*args):
    (flats, (ce, re), (lin_out, a2_out, b2_out, neg_out),
     (dv, cidx_v, ridx_v, crows_v, rrows_v, lin_v, a2_v, b2_v, neg_v,
      sem)) = (args[:7], args[7:9], args[9:13], args[13:])
    wid = lax.axis_index("s") * NC + lax.axis_index("c")
    descs = []
    offs = []
    off = 0
    for t, f in enumerate(flats):
        n = 16 * _TBL_COLS[t]
        descs.append(pltpu.async_copy(
            f.at[pl.ds(wid * n, n)], dv.at[pl.ds(off, n)], sem))
        offs.append(off)
        off += n
    for d in descs:
        d.wait()
    iota = lax.iota(jnp.int32, 16)
    for t, kind, col, sec in _GROUPS:
        idx16 = plsc.load_gather(dv, [iota * _TBL_COLS[t] + (offs[t] + col)])
        if kind == 'c':
            cidx_v[pl.ds(sec * 16, 16)] = idx16
        else:
            ridx_v[pl.ds(sec * 16, 16)] = idx16
    half = (N_CSEC * 16) // 2
    for j in range(2):
        pltpu.async_copy(ce.at[cidx_v.at[pl.ds(j * half, half)]],
                         crows_v.at[pl.ds(j * half, half)], sem).wait()
    pltpu.async_copy(re.at[ridx_v], rrows_v, sem).wait()

    def crow(sec, j, ch):
        return crows_v[sec * 16 + j, pl.ds(ch * 16, 16)]

    def rrow(sec, j, ch):
        return rrows_v[sec * 16 + j, pl.ds(ch * 16, 16)]

    def relu(x):
        return jnp.maximum(x, 0.0)

    def body(j, lin_acc):
        zero = jnp.zeros((16,), jnp.float32)
        for ch in range(8):
            cc, co = crow(0, j, ch), jnp.abs(crow(0, j, ch + 8))
            dc, do = crow(1, j, ch), jnp.abs(crow(1, j, ch + 8))
            t1 = relu(jnp.abs(cc - dc) + co - do)
            cc, co = crow(5, j, ch), jnp.abs(crow(5, j, ch + 8))
            dc, do = crow(6, j, ch), jnp.abs(crow(6, j, ch + 8))
            t3 = relu(jnp.abs(cc + rrow(0, j, ch) - dc) + co - do)
            cc, co = crow(7, j, ch), jnp.abs(crow(7, j, ch + 8))
            dc, do = crow(8, j, ch), jnp.abs(crow(8, j, ch + 8))
            t4 = relu(jnp.abs(cc - rrow(1, j, ch) - dc) + co - do)
            cc, co = crow(9, j, ch), jnp.abs(crow(9, j, ch + 8))
            dc, do = crow(10, j, ch), jnp.abs(crow(10, j, ch + 8))
            td = relu(-jnp.abs(cc - dc) + co + do)
            lin_acc = lin_acc + t1 * t1 + t3 * t3 + t4 * t4 + td * td
        a2 = zero
        b2 = zero
        for ch in range(8):
            cc, co = crow(2, j, ch), jnp.abs(crow(2, j, ch + 8))
            dc, do = crow(3, j, ch), jnp.abs(crow(3, j, ch + 8))
            ec, eo = crow(4, j, ch), jnp.abs(crow(4, j, ch + 8))
            lower = jnp.maximum(cc - co, dc - do)
            upper = jnp.minimum(cc + co, dc + do)
            ic = (lower + upper) * 0.5
            io = jnp.abs(upper - lower) * 0.5
            ta = relu(jnp.abs(ic - ec) + io - eo)
            tb = relu(lower - upper)
            a2 = a2 + ta * ta
            b2 = b2 + tb * tb
        a2_v[j, pl.ds(0, 16)] = a2
        b2_v[j, pl.ds(0, 16)] = b2
        for h, (sc_, sd, sr) in enumerate(((11, 13, 2), (12, 14, 3))):
            n2 = zero
            for ch in range(8):
                cc, co = crow(sc_, j, ch), jnp.abs(crow(sc_, j, ch + 8))
                dc, do = crow(sd, j, ch), jnp.abs(crow(sd, j, ch + 8))
                tn = relu(jnp.abs(cc + rrow(sr, j, ch) - dc) - co - do)
                n2 = n2 + tn * tn
            neg_v[h * 16 + j, pl.ds(0, 16)] = n2
        return lin_acc

    lin_acc = lax.fori_loop(0, 16, body, jnp.zeros((16,), jnp.float32))
    lin_v[pl.ds(0, 16)] = lin_acc

    pltpu.sync_copy(lin_v, lin_out.at[wid])
    pltpu.sync_copy(a2_v, a2_out.at[wid])
    pltpu.sync_copy(b2_v, b2_out.at[wid])
    pltpu.sync_copy(neg_v, neg_out.at[wid])


def _gather_sc(cols, class_embeds, relation_embeds):
    mesh = plsc.VectorSubcoreMesh(core_axis_name="c", subcore_axis_name="s",
                                  num_cores=NC, num_subcores=NS)
    f = pl.kernel(
        _gather_body,
        out_type=[
            jax.ShapeDtypeStruct((NW, 16), jnp.float32),
            jax.ShapeDtypeStruct((NW, 16, 16), jnp.float32),
            jax.ShapeDtypeStruct((NW, 16, 16), jnp.float32),
            jax.ShapeDtypeStruct((NW, 32, 16), jnp.float32),
        ],
        mesh=mesh,
        compiler_params=pltpu.CompilerParams(needs_layout_passes=False),
        scratch_types=[
            pltpu.VMEM((16 * sum(_TBL_COLS),), jnp.int32),
            pltpu.VMEM((N_CSEC * 16,), jnp.int32),
            pltpu.VMEM((N_RSEC * 16,), jnp.int32),
            pltpu.VMEM((N_CSEC * 16, 2 * DIM), jnp.float32),
            pltpu.VMEM((N_RSEC * 16, DIM), jnp.float32),
            pltpu.VMEM((16,), jnp.float32),
            pltpu.VMEM((16, 16), jnp.float32),
            pltpu.VMEM((16, 16), jnp.float32),
            pltpu.VMEM((32, 16), jnp.float32),
            pltpu.SemaphoreType.DMA,
        ],
    )
    return f(*cols, class_embeds, relation_embeds)


def _epi_body(lin_ref, a2_ref, b2_ref, neg_ref, out_ref):
    loss_lin = jnp.sum(lin_ref[...]) / BATCH
    a2 = jnp.sum(a2_ref[...], axis=1)
    b2 = jnp.sum(b2_ref[...], axis=1)
    a = jnp.sqrt(a2)
    b = jnp.sqrt(b2)
    loss2 = (jnp.mean(a2) + jnp.mean(b2) + 2.0 * jnp.mean(a) * jnp.mean(b))
    n2 = jnp.sum(neg_ref[...], axis=1)
    n = jnp.sqrt(n2)
    neg_loss = jnp.mean(jnp.square(NEG_DIST - n))
    total = loss_lin + loss2 + neg_loss
    out_ref[...] = jnp.reshape(total, (1, 1))


def _epilogue_tc(lin, a2, b2, neg):
    return pl.pallas_call(
        _epi_body,
        out_shape=jax.ShapeDtypeStruct((1, 1), jnp.float32),
    )(lin, a2.reshape(NW * 16, 16), b2.reshape(NW * 16, 16),
      neg.reshape(NW * 32, 16))


def kernel(nf1, nf2, nf3, nf4, disjoint, nf3_neg0, nf3_neg1,
           class_embeds, relation_embeds):
    tables = (nf1, nf2, nf3, nf4, disjoint, nf3_neg0, nf3_neg1)
    flats = []
    for t in range(7):
        iv = _IVECS[(5 if t == 6 else t)]
        c = _TBL_COLS[t]
        ir = np.repeat(iv, c)
        ic = np.tile(np.arange(c, dtype=np.int32), BATCH)
        flats.append(tables[t][ir, ic])
    lin, a2, b2, neg = _gather_sc(flats, class_embeds, relation_embeds)
    return _epilogue_tc(lin, a2, b2, neg)[0, 0]


# speedup vs baseline: 6.4004x; 1.1978x over previous
import numpy as np
import jax
import jax.numpy as jnp
from jax import lax
from jax.experimental import pallas as pl
from jax.experimental.pallas import tpu as pltpu
from jax.experimental.pallas import tpu_sc as plsc

DIM = 128
BATCH = 512
NEG_DIST = 2.0

NC = 2
NS = 16
NW = NC * NS

N_CSEC = 15
N_RSEC = 4
NCH = (2 * DIM) // 16

_TBL_COLS = (2, 3, 3, 3, 2, 3, 3)
_GROUPS = (
    (0, 'c', 0, 0), (0, 'c', 1, 1),
    (1, 'c', 0, 2), (1, 'c', 1, 3), (1, 'c', 2, 4),
    (2, 'c', 0, 5), (2, 'c', 2, 6), (2, 'r', 1, 0),
    (3, 'c', 1, 7), (3, 'c', 2, 8), (3, 'r', 0, 1),
    (4, 'c', 0, 9), (4, 'c', 1, 10),
    (5, 'c', 0, 11), (5, 'c', 2, 13), (5, 'r', 1, 2),
    (6, 'c', 0, 12), (6, 'c', 2, 14), (6, 'r', 1, 3),
)


_U32 = np.uint32


def _rol(x, d):
    return (x << _U32(d)) | (x >> _U32(32 - d))


def _threefry2x32(k1, k2, x1, x2):
    ks0, ks1 = _U32(k1), _U32(k2)
    ks2 = ks0 ^ ks1 ^ _U32(0x1BD11BDA)
    with np.errstate(over='ignore'):
        x0 = (np.asarray(x1, _U32) + ks0).astype(_U32)
        y1 = (np.asarray(x2, _U32) + ks1).astype(_U32)

        def rounds(x0, y1, rots):
            for r in rots:
                x0 = (x0 + y1).astype(_U32)
                y1 = x0 ^ _rol(y1, r)
            return x0, y1

        x0, y1 = rounds(x0, y1, (13, 15, 26, 6))
        x0 = (x0 + ks1).astype(_U32); y1 = (y1 + ks2 + _U32(1)).astype(_U32)
        x0, y1 = rounds(x0, y1, (17, 29, 16, 24))
        x0 = (x0 + ks2).astype(_U32); y1 = (y1 + ks0 + _U32(2)).astype(_U32)
        x0, y1 = rounds(x0, y1, (13, 15, 26, 6))
        x0 = (x0 + ks0).astype(_U32); y1 = (y1 + ks1 + _U32(3)).astype(_U32)
        x0, y1 = rounds(x0, y1, (17, 29, 16, 24))
        x0 = (x0 + ks1).astype(_U32); y1 = (y1 + ks2 + _U32(4)).astype(_U32)
        x0, y1 = rounds(x0, y1, (13, 15, 26, 6))
        x0 = (x0 + ks2).astype(_U32); y1 = (y1 + ks0 + _U32(5)).astype(_U32)
    return x0, y1


def _np_split(key, num):
    b1, b2 = _threefry2x32(key[0], key[1],
                           np.zeros(num, _U32), np.arange(num, dtype=_U32))
    return [(b1[i], b2[i]) for i in range(num)]


def _np_bits(key, n):
    b1, b2 = _threefry2x32(key[0], key[1],
                           np.zeros(n, _U32), np.arange(n, dtype=_U32))
    return b1 ^ b2


def _np_randint(key, n, minval, maxval):
    k1, k2 = _np_split(key, 2)
    higher, lower = _np_bits(k1, n), _np_bits(k2, n)
    span = _U32(maxval - minval)
    with np.errstate(over='ignore'):
        mult = _U32(int(2 ** 16) % int(span))
        mult = _U32((int(mult) * int(mult)) % (2 ** 32)) % span
        off = ((higher % span) * mult + lower % span).astype(_U32) % span
    return (np.int32(minval) + off.astype(np.int32)).astype(np.int32)


_IVECS = [_np_randint(k, BATCH, 0, 100000)
          for k in _np_split((_U32(0), _U32(42)), 6)]


def _gather_body(*args):
    (flats, (ce, re), (lin_out, a2_out, b2_out, neg_out),
     (dv, cidx_v, ridx_v, crows_v, rrows_v, lin_v, a2_v, b2_v, neg_v,
      sem)) = (args[:7], args[7:9], args[9:13], args[13:])
    wid = lax.axis_index("s") * NC + lax.axis_index("c")
    descs = []
    offs = []
    off = 0
    for t, f in enumerate(flats):
        n = 16 * _TBL_COLS[t]
        descs.append(pltpu.async_copy(
            f.at[pl.ds(wid * n, n)], dv.at[pl.ds(off, n)], sem))
        offs.append(off)
        off += n
    for d in descs:
        d.wait()
    iota = lax.iota(jnp.int32, 16)
    for t, kind, col, sec in _GROUPS:
        idx16 = plsc.load_gather(dv, [iota * _TBL_COLS[t] + (offs[t] + col)])
        if kind == 'c':
            cidx_v[pl.ds(sec * 16, 16)] = idx16
        else:
            ridx_v[pl.ds(sec * 16, 16)] = idx16
    half = (N_CSEC * 16) // 2
    for j in range(2):
        pltpu.async_copy(ce.at[cidx_v.at[pl.ds(j * half, half)]],
                         crows_v.at[pl.ds(j * half, half)], sem).wait()
    pltpu.async_copy(re.at[ridx_v], rrows_v, sem).wait()

    def crow(sec, j, ch):
        return crows_v[sec * 16 + j, pl.ds(ch * 16, 16)]

    def rrow(sec, j, ch):
        return rrows_v[sec * 16 + j, pl.ds(ch * 16, 16)]

    def relu(x):
        return jnp.maximum(x, 0.0)

    def body(j, lin_acc):
        zero = jnp.zeros((16,), jnp.float32)
        for ch in range(8):
            cc, co = crow(0, j, ch), jnp.abs(crow(0, j, ch + 8))
            dc, do = crow(1, j, ch), jnp.abs(crow(1, j, ch + 8))
            t1 = relu(jnp.abs(cc - dc) + co - do)
            cc, co = crow(5, j, ch), jnp.abs(crow(5, j, ch + 8))
            dc, do = crow(6, j, ch), jnp.abs(crow(6, j, ch + 8))
            t3 = relu(jnp.abs(cc + rrow(0, j, ch) - dc) + co - do)
            cc, co = crow(7, j, ch), jnp.abs(crow(7, j, ch + 8))
            dc, do = crow(8, j, ch), jnp.abs(crow(8, j, ch + 8))
            t4 = relu(jnp.abs(cc - rrow(1, j, ch) - dc) + co - do)
            cc, co = crow(9, j, ch), jnp.abs(crow(9, j, ch + 8))
            dc, do = crow(10, j, ch), jnp.abs(crow(10, j, ch + 8))
            td = relu(-jnp.abs(cc - dc) + co + do)
            lin_acc = lin_acc + t1 * t1 + t3 * t3 + t4 * t4 + td * td
        a2 = zero
        b2 = zero
        for ch in range(8):
            cc, co = crow(2, j, ch), jnp.abs(crow(2, j, ch + 8))
            dc, do = crow(3, j, ch), jnp.abs(crow(3, j, ch + 8))
            ec, eo = crow(4, j, ch), jnp.abs(crow(4, j, ch + 8))
            lower = jnp.maximum(cc - co, dc - do)
            upper = jnp.minimum(cc + co, dc + do)
            ic = (lower + upper) * 0.5
            io = jnp.abs(upper - lower) * 0.5
            ta = relu(jnp.abs(ic - ec) + io - eo)
            tb = relu(lower - upper)
            a2 = a2 + ta * ta
            b2 = b2 + tb * tb
        a2_v[j, pl.ds(0, 16)] = a2
        b2_v[j, pl.ds(0, 16)] = b2
        for h, (sc_, sd, sr) in enumerate(((11, 13, 2), (12, 14, 3))):
            n2 = zero
            for ch in range(8):
                cc, co = crow(sc_, j, ch), jnp.abs(crow(sc_, j, ch + 8))
                dc, do = crow(sd, j, ch), jnp.abs(crow(sd, j, ch + 8))
                tn = relu(jnp.abs(cc + rrow(sr, j, ch) - dc) - co - do)
                n2 = n2 + tn * tn
            neg_v[h * 16 + j, pl.ds(0, 16)] = n2
        return lin_acc

    lin_acc = lax.fori_loop(0, 16, body, jnp.zeros((16,), jnp.float32))
    lin_v[pl.ds(0, 16)] = lin_acc

    pltpu.sync_copy(lin_v, lin_out.at[wid])
    pltpu.sync_copy(a2_v, a2_out.at[wid])
    pltpu.sync_copy(b2_v, b2_out.at[wid])
    pltpu.sync_copy(neg_v, neg_out.at[wid])


def _gather_sc(cols, class_embeds, relation_embeds):
    mesh = plsc.VectorSubcoreMesh(core_axis_name="c", subcore_axis_name="s",
                                  num_cores=NC, num_subcores=NS)
    f = pl.kernel(
        _gather_body,
        out_type=[
            jax.ShapeDtypeStruct((NW, 16), jnp.float32),
            jax.ShapeDtypeStruct((NW, 16, 16), jnp.float32),
            jax.ShapeDtypeStruct((NW, 16, 16), jnp.float32),
            jax.ShapeDtypeStruct((NW, 32, 16), jnp.float32),
        ],
        mesh=mesh,
        compiler_params=pltpu.CompilerParams(needs_layout_passes=False),
        scratch_types=[
            pltpu.VMEM((16 * sum(_TBL_COLS),), jnp.int32),
            pltpu.VMEM((N_CSEC * 16,), jnp.int32),
            pltpu.VMEM((N_RSEC * 16,), jnp.int32),
            pltpu.VMEM((N_CSEC * 16, 2 * DIM), jnp.float32),
            pltpu.VMEM((N_RSEC * 16, DIM), jnp.float32),
            pltpu.VMEM((16,), jnp.float32),
            pltpu.VMEM((16, 16), jnp.float32),
            pltpu.VMEM((16, 16), jnp.float32),
            pltpu.VMEM((32, 16), jnp.float32),
            pltpu.SemaphoreType.DMA,
        ],
    )
    return f(*cols, class_embeds, relation_embeds)


def _epi_body(lin_ref, a2_ref, b2_ref, neg_ref, out_ref):
    loss_lin = jnp.sum(lin_ref[...]) / BATCH
    a2 = jnp.sum(a2_ref[...], axis=1)
    b2 = jnp.sum(b2_ref[...], axis=1)
    a = jnp.sqrt(a2)
    b = jnp.sqrt(b2)
    loss2 = (jnp.mean(a2) + jnp.mean(b2) + 2.0 * jnp.mean(a) * jnp.mean(b))
    n2 = jnp.sum(neg_ref[...], axis=1)
    n = jnp.sqrt(n2)
    neg_loss = jnp.mean(jnp.square(NEG_DIST - n))
    total = loss_lin + loss2 + neg_loss
    out_ref[...] = jnp.reshape(total, (1, 1))


def _epilogue_tc(lin, a2, b2, neg):
    return pl.pallas_call(
        _epi_body,
        out_shape=jax.ShapeDtypeStruct((1, 1), jnp.float32),
    )(lin, a2.reshape(NW * 16, 16), b2.reshape(NW * 16, 16),
      neg.reshape(NW * 32, 16))


def kernel(nf1, nf2, nf3, nf4, disjoint, nf3_neg0, nf3_neg1,
           class_embeds, relation_embeds):
    tables = (nf1, nf2, nf3, nf4, disjoint, nf3_neg0, nf3_neg1)
    flats = []
    for t in range(7):
        iv = _IVECS[(5 if t == 6 else t)]
        c = _TBL_COLS[t]
        ir = np.repeat(iv, c)
        ic = np.tile(np.arange(c, dtype=np.int32), BATCH)
        if ir.size <= 1024:
            pad = 1040 - ir.size
            ir = np.concatenate([ir, np.zeros(pad, ir.dtype)])
            ic = np.concatenate([ic, np.zeros(pad, ic.dtype)])
        flats.append(tables[t][ir, ic])
    lin, a2, b2, neg = _gather_sc(flats, class_embeds, relation_embeds)
    return _epilogue_tc(lin, a2, b2, neg)[0, 0]


# speedup vs baseline: 6.5230x; 1.0192x over previous
import numpy as np
import jax
import jax.numpy as jnp
from jax import lax
from jax.experimental import pallas as pl
from jax.experimental.pallas import tpu as pltpu
from jax.experimental.pallas import tpu_sc as plsc

DIM = 128
BATCH = 512
NEG_DIST = 2.0

NC = 2
NS = 16
NW = NC * NS

N_CSEC = 15
N_RSEC = 4
NCH = (2 * DIM) // 16

_TBL_COLS = (2, 3, 3, 3, 2, 3, 3)
_GROUPS = (
    (0, 'c', 0, 0), (0, 'c', 1, 1),
    (1, 'c', 0, 2), (1, 'c', 1, 3), (1, 'c', 2, 4),
    (2, 'c', 0, 5), (2, 'c', 2, 6), (2, 'r', 1, 0),
    (3, 'c', 1, 7), (3, 'c', 2, 8), (3, 'r', 0, 1),
    (4, 'c', 0, 9), (4, 'c', 1, 10),
    (5, 'c', 0, 11), (5, 'c', 2, 13), (5, 'r', 1, 2),
    (6, 'c', 0, 12), (6, 'c', 2, 14), (6, 'r', 1, 3),
)


_U32 = np.uint32


def _rol(x, d):
    return (x << _U32(d)) | (x >> _U32(32 - d))


def _threefry2x32(k1, k2, x1, x2):
    ks0, ks1 = _U32(k1), _U32(k2)
    ks2 = ks0 ^ ks1 ^ _U32(0x1BD11BDA)
    with np.errstate(over='ignore'):
        x0 = (np.asarray(x1, _U32) + ks0).astype(_U32)
        y1 = (np.asarray(x2, _U32) + ks1).astype(_U32)

        def rounds(x0, y1, rots):
            for r in rots:
                x0 = (x0 + y1).astype(_U32)
                y1 = x0 ^ _rol(y1, r)
            return x0, y1

        x0, y1 = rounds(x0, y1, (13, 15, 26, 6))
        x0 = (x0 + ks1).astype(_U32); y1 = (y1 + ks2 + _U32(1)).astype(_U32)
        x0, y1 = rounds(x0, y1, (17, 29, 16, 24))
        x0 = (x0 + ks2).astype(_U32); y1 = (y1 + ks0 + _U32(2)).astype(_U32)
        x0, y1 = rounds(x0, y1, (13, 15, 26, 6))
        x0 = (x0 + ks0).astype(_U32); y1 = (y1 + ks1 + _U32(3)).astype(_U32)
        x0, y1 = rounds(x0, y1, (17, 29, 16, 24))
        x0 = (x0 + ks1).astype(_U32); y1 = (y1 + ks2 + _U32(4)).astype(_U32)
        x0, y1 = rounds(x0, y1, (13, 15, 26, 6))
        x0 = (x0 + ks2).astype(_U32); y1 = (y1 + ks0 + _U32(5)).astype(_U32)
    return x0, y1


def _np_split(key, num):
    b1, b2 = _threefry2x32(key[0], key[1],
                           np.zeros(num, _U32), np.arange(num, dtype=_U32))
    return [(b1[i], b2[i]) for i in range(num)]


def _np_bits(key, n):
    b1, b2 = _threefry2x32(key[0], key[1],
                           np.zeros(n, _U32), np.arange(n, dtype=_U32))
    return b1 ^ b2


def _np_randint(key, n, minval, maxval):
    k1, k2 = _np_split(key, 2)
    higher, lower = _np_bits(k1, n), _np_bits(k2, n)
    span = _U32(maxval - minval)
    with np.errstate(over='ignore'):
        mult = _U32(int(2 ** 16) % int(span))
        mult = _U32((int(mult) * int(mult)) % (2 ** 32)) % span
        off = ((higher % span) * mult + lower % span).astype(_U32) % span
    return (np.int32(minval) + off.astype(np.int32)).astype(np.int32)


_IVECS = [_np_randint(k, BATCH, 0, 100000)
          for k in _np_split((_U32(0), _U32(42)), 6)]


def _gather_body(*args):
    (flats, (ce, re), (lin_out, a2_out, b2_out, neg_out),
     (dv, cidx_v, ridx_v, crows_v, rrows_v, lin_v, a2_v, b2_v, neg_v,
      sem)) = (args[:7], args[7:9], args[9:13], args[13:])
    wid = lax.axis_index("s") * NC + lax.axis_index("c")
    descs = []
    offs = []
    off = 0
    for t, f in enumerate(flats):
        n = 16 * _TBL_COLS[t]
        descs.append(pltpu.async_copy(
            f.at[pl.ds(wid * n, n)], dv.at[pl.ds(off, n)], sem))
        offs.append(off)
        off += n
    for d in descs:
        d.wait()
    iota = lax.iota(jnp.int32, 16)
    for t, kind, col, sec in _GROUPS:
        idx16 = plsc.load_gather(dv, [iota * _TBL_COLS[t] + (offs[t] + col)])
        if kind == 'c':
            cidx_v[pl.ds(sec * 16, 16)] = idx16
        else:
            ridx_v[pl.ds(sec * 16, 16)] = idx16
    half = (N_CSEC * 16) // 2
    descs = []
    for j in range(2):
        descs.append(pltpu.async_copy(
            ce.at[cidx_v.at[pl.ds(j * half, half)]],
            crows_v.at[pl.ds(j * half, half)], sem))
    descs.append(pltpu.async_copy(re.at[ridx_v], rrows_v, sem))
    for d in descs:
        d.wait()

    def crow(sec, j, ch):
        return crows_v[sec * 16 + j, pl.ds(ch * 16, 16)]

    def rrow(sec, j, ch):
        return rrows_v[sec * 16 + j, pl.ds(ch * 16, 16)]

    def relu(x):
        return jnp.maximum(x, 0.0)

    def body(j, lin_acc):
        zero = jnp.zeros((16,), jnp.float32)
        for ch in range(8):
            cc, co = crow(0, j, ch), jnp.abs(crow(0, j, ch + 8))
            dc, do = crow(1, j, ch), jnp.abs(crow(1, j, ch + 8))
            t1 = relu(jnp.abs(cc - dc) + co - do)
            cc, co = crow(5, j, ch), jnp.abs(crow(5, j, ch + 8))
            dc, do = crow(6, j, ch), jnp.abs(crow(6, j, ch + 8))
            t3 = relu(jnp.abs(cc + rrow(0, j, ch) - dc) + co - do)
            cc, co = crow(7, j, ch), jnp.abs(crow(7, j, ch + 8))
            dc, do = crow(8, j, ch), jnp.abs(crow(8, j, ch + 8))
            t4 = relu(jnp.abs(cc - rrow(1, j, ch) - dc) + co - do)
            cc, co = crow(9, j, ch), jnp.abs(crow(9, j, ch + 8))
            dc, do = crow(10, j, ch), jnp.abs(crow(10, j, ch + 8))
            td = relu(-jnp.abs(cc - dc) + co + do)
            lin_acc = lin_acc + t1 * t1 + t3 * t3 + t4 * t4 + td * td
        a2 = zero
        b2 = zero
        for ch in range(8):
            cc, co = crow(2, j, ch), jnp.abs(crow(2, j, ch + 8))
            dc, do = crow(3, j, ch), jnp.abs(crow(3, j, ch + 8))
            ec, eo = crow(4, j, ch), jnp.abs(crow(4, j, ch + 8))
            lower = jnp.maximum(cc - co, dc - do)
            upper = jnp.minimum(cc + co, dc + do)
            ic = (lower + upper) * 0.5
            io = jnp.abs(upper - lower) * 0.5
            ta = relu(jnp.abs(ic - ec) + io - eo)
            tb = relu(lower - upper)
            a2 = a2 + ta * ta
            b2 = b2 + tb * tb
        a2_v[j, pl.ds(0, 16)] = a2
        b2_v[j, pl.ds(0, 16)] = b2
        for h, (sc_, sd, sr) in enumerate(((11, 13, 2), (12, 14, 3))):
            n2 = zero
            for ch in range(8):
                cc, co = crow(sc_, j, ch), jnp.abs(crow(sc_, j, ch + 8))
                dc, do = crow(sd, j, ch), jnp.abs(crow(sd, j, ch + 8))
                tn = relu(jnp.abs(cc + rrow(sr, j, ch) - dc) - co - do)
                n2 = n2 + tn * tn
            neg_v[h * 16 + j, pl.ds(0, 16)] = n2
        return lin_acc

    lin_acc = lax.fori_loop(0, 16, body, jnp.zeros((16,), jnp.float32))
    lin_v[pl.ds(0, 16)] = lin_acc

    pltpu.sync_copy(lin_v, lin_out.at[wid])
    pltpu.sync_copy(a2_v, a2_out.at[wid])
    pltpu.sync_copy(b2_v, b2_out.at[wid])
    pltpu.sync_copy(neg_v, neg_out.at[wid])


def _gather_sc(cols, class_embeds, relation_embeds):
    mesh = plsc.VectorSubcoreMesh(core_axis_name="c", subcore_axis_name="s",
                                  num_cores=NC, num_subcores=NS)
    f = pl.kernel(
        _gather_body,
        out_type=[
            jax.ShapeDtypeStruct((NW, 16), jnp.float32),
            jax.ShapeDtypeStruct((NW, 16, 16), jnp.float32),
            jax.ShapeDtypeStruct((NW, 16, 16), jnp.float32),
            jax.ShapeDtypeStruct((NW, 32, 16), jnp.float32),
        ],
        mesh=mesh,
        compiler_params=pltpu.CompilerParams(needs_layout_passes=False),
        scratch_types=[
            pltpu.VMEM((16 * sum(_TBL_COLS),), jnp.int32),
            pltpu.VMEM((N_CSEC * 16,), jnp.int32),
            pltpu.VMEM((N_RSEC * 16,), jnp.int32),
            pltpu.VMEM((N_CSEC * 16, 2 * DIM), jnp.float32),
            pltpu.VMEM((N_RSEC * 16, DIM), jnp.float32),
            pltpu.VMEM((16,), jnp.float32),
            pltpu.VMEM((16, 16), jnp.float32),
            pltpu.VMEM((16, 16), jnp.float32),
            pltpu.VMEM((32, 16), jnp.float32),
            pltpu.SemaphoreType.DMA,
        ],
    )
    return f(*cols, class_embeds, relation_embeds)


def _epi_body(lin_ref, a2_ref, b2_ref, neg_ref, out_ref):
    loss_lin = jnp.sum(lin_ref[...]) / BATCH
    a2 = jnp.sum(a2_ref[...], axis=1)
    b2 = jnp.sum(b2_ref[...], axis=1)
    a = jnp.sqrt(a2)
    b = jnp.sqrt(b2)
    loss2 = (jnp.mean(a2) + jnp.mean(b2) + 2.0 * jnp.mean(a) * jnp.mean(b))
    n2 = jnp.sum(neg_ref[...], axis=1)
    n = jnp.sqrt(n2)
    neg_loss = jnp.mean(jnp.square(NEG_DIST - n))
    total = loss_lin + loss2 + neg_loss
    out_ref[...] = jnp.reshape(total, (1, 1))


def _epilogue_tc(lin, a2, b2, neg):
    return pl.pallas_call(
        _epi_body,
        out_shape=jax.ShapeDtypeStruct((1, 1), jnp.float32),
    )(lin, a2.reshape(NW * 16, 16), b2.reshape(NW * 16, 16),
      neg.reshape(NW * 32, 16))


def kernel(nf1, nf2, nf3, nf4, disjoint, nf3_neg0, nf3_neg1,
           class_embeds, relation_embeds):
    tables = (nf1, nf2, nf3, nf4, disjoint, nf3_neg0, nf3_neg1)
    flats = []
    for t in range(7):
        iv = _IVECS[(5 if t == 6 else t)]
        c = _TBL_COLS[t]
        ir = np.repeat(iv, c)
        ic = np.tile(np.arange(c, dtype=np.int32), BATCH)
        if ir.size <= 1024:
            pad = 1040 - ir.size
            ir = np.concatenate([ir, np.zeros(pad, ir.dtype)])
            ic = np.concatenate([ic, np.zeros(pad, ic.dtype)])
        flats.append(tables[t][ir, ic])
    lin, a2, b2, neg = _gather_sc(flats, class_embeds, relation_embeds)
    return _epilogue_tc(lin, a2, b2, neg)[0, 0]
